# Initial kernel scaffold; baseline (speedup 1.0000x reference)
import functools

import jax
import jax.numpy as jnp
from jax import lax
from jax.experimental import pallas as pl
from jax.experimental.pallas import tpu as pltpu
from jax.experimental.pallas import tpu_sc as plsc

B = 16384
L = 50
D = 32
NCAT = 128
NC, NS = 2, 16
NW = NC * NS
RPW = B // NW
C = 16
NCH = RPW // C
GI = C * L
IPW = RPW * L


def _sc_pool(desc_flat, table):
    mesh = plsc.VectorSubcoreMesh(core_axis_name="c", subcore_axis_name="s")

    @functools.partial(
        pl.kernel,
        out_type=jax.ShapeDtypeStruct((B, D), jnp.float32),
        mesh=mesh,
        scratch_types=[
            pltpu.VMEM((IPW,), jnp.int32),
            pltpu.VMEM((GI, D), jnp.float32),
            pltpu.VMEM((GI, D), jnp.float32),
            pltpu.VMEM((RPW, D), jnp.float32),
            pltpu.SemaphoreType.DMA,
            pltpu.SemaphoreType.DMA,
        ],
    )
    def k(desc_hbm, table_hbm, out_hbm, idx_v, rows0, rows1, sums_v, sem0, sem1):
        cid = lax.axis_index("c")
        sid = lax.axis_index("s")
        wid = sid * NC + cid
        ibase = wid * IPW

        pltpu.sync_copy(desc_hbm.at[pl.ds(ibase, IPW)], idx_v)

        bufs = (rows0, rows1)
        sems = (sem0, sem1)

        def fire(c, sub):
            pltpu.async_copy(
                table_hbm.at[idx_v.at[pl.ds(c * GI, GI)]], bufs[sub], sems[sub]
            )

        def wait(c, sub):
            pltpu.make_async_copy(
                table_hbm.at[idx_v.at[pl.ds(c * GI, GI)]], bufs[sub], sems[sub]
            ).wait()

        fire(0, 0)
        fire(1, 1)

        def reduce_chunk(c, rows):
            def jbody(j, carry):
                r0 = j * L
                lo = []
                hi = []
                for g in range(5):
                    sl = rows[r0 + g, pl.ds(0, 16)]
                    sh = rows[r0 + g, pl.ds(16, 16)]
                    for l in range(g + 5, L, 5):
                        sl = sl + rows[r0 + l, pl.ds(0, 16)]
                        sh = sh + rows[r0 + l, pl.ds(16, 16)]
                    lo.append(sl)
                    hi.append(sh)
                out_r = c * C + j
                sums_v[out_r, pl.ds(0, 16)] = (lo[0] + lo[1]) + (lo[2] + lo[3]) + lo[4]
                sums_v[out_r, pl.ds(16, 16)] = (hi[0] + hi[1]) + (hi[2] + hi[3]) + hi[4]
                return carry

            lax.fori_loop(0, C, jbody, 0)

        def pbody(p, carry):
            for sub in range(2):
                c = p * 2 + sub
                wait(c, sub)
                reduce_chunk(c, bufs[sub])
                nxt = c + 2

                @pl.when(nxt < NCH)
                def _():
                    fire(nxt, sub)

            return carry

        lax.fori_loop(0, NCH // 2, pbody, 0)
        pltpu.sync_copy(sums_v, out_hbm.at[pl.ds(wid * RPW, RPW)])

    return k(desc_flat, table)


def _lin_body(s_ref, a_ref, wm_ref, wa_ref, b_ref, o_ref):
    o_ref[...] = (
        jnp.dot(s_ref[...], wm_ref[...], preferred_element_type=jnp.float32)
        + a_ref[...] * wa_ref[...]
        + b_ref[...]
    )


def _tc_linear(sums, amounts, wm, wa, b2):
    blk = 1024
    return pl.pallas_call(
        _lin_body,
        grid=(B // blk,),
        in_specs=[
            pl.BlockSpec((blk, D), lambda i: (i, 0)),
            pl.BlockSpec((blk, 1), lambda i: (i, 0)),
            pl.BlockSpec((D, NCAT), lambda i: (0, 0)),
            pl.BlockSpec((1, NCAT), lambda i: (0, 0)),
            pl.BlockSpec((1, NCAT), lambda i: (0, 0)),
        ],
        out_specs=pl.BlockSpec((blk, NCAT), lambda i: (i, 0)),
        out_shape=jax.ShapeDtypeStruct((B, NCAT), jnp.float32),
    )(sums, amounts, wm, wa, b2)


def kernel(descriptions, amounts, table, W, b):
    desc_flat = descriptions.reshape(-1)
    sums = _sc_pool(desc_flat, table)
    wm = W[:D] * (1.0 / L)
    wa = W[D : D + 1]
    b2 = b.reshape(1, NCAT)
    return _tc_linear(sums, amounts, wm, wa, b2)


# speedup vs baseline: 2.8890x; 2.8890x over previous
import functools

import jax
import jax.numpy as jnp
from jax import lax
from jax.experimental import pallas as pl
from jax.experimental.pallas import tpu as pltpu
from jax.experimental.pallas import tpu_sc as plsc

B = 16384
L = 50
D = 32
NCAT = 128
NC, NS = 2, 16
NW = NC * NS
RPW = B // NW
C = 16
NCH = RPW // C
GI = C * L
IPW = RPW * L


def _sc_pool(desc_flat, table):
    mesh = plsc.VectorSubcoreMesh(core_axis_name="c", subcore_axis_name="s")

    @functools.partial(
        pl.kernel,
        out_type=jax.ShapeDtypeStruct((B, D), jnp.float32),
        mesh=mesh,
        compiler_params=pltpu.CompilerParams(use_tc_tiling_on_sc=False),
        scratch_types=[
            pltpu.VMEM((IPW,), jnp.int32),
            pltpu.VMEM((GI, D), jnp.float32),
            pltpu.VMEM((GI, D), jnp.float32),
            pltpu.VMEM((RPW, D), jnp.float32),
            pltpu.SemaphoreType.DMA,
            pltpu.SemaphoreType.DMA,
        ],
    )
    def k(desc_hbm, table_hbm, out_hbm, idx_v, rows0, rows1, sums_v, sem0, sem1):
        cid = lax.axis_index("c")
        sid = lax.axis_index("s")
        wid = sid * NC + cid
        ibase = wid * IPW

        pltpu.sync_copy(desc_hbm.at[pl.ds(ibase, IPW)], idx_v)

        bufs = (rows0, rows1)
        sems = (sem0, sem1)

        def fire(c, sub):
            pltpu.async_copy(
                table_hbm.at[idx_v.at[pl.ds(c * GI, GI)]], bufs[sub], sems[sub]
            )

        def wait(c, sub):
            pltpu.make_async_copy(
                table_hbm.at[idx_v.at[pl.ds(c * GI, GI)]], bufs[sub], sems[sub]
            ).wait()

        fire(0, 0)
        fire(1, 1)

        def reduce_chunk(c, rows):
            def jbody(j, carry):
                r0 = j * L
                lo = []
                hi = []
                for g in range(5):
                    sl = rows[r0 + g, pl.ds(0, 16)]
                    sh = rows[r0 + g, pl.ds(16, 16)]
                    for l in range(g + 5, L, 5):
                        sl = sl + rows[r0 + l, pl.ds(0, 16)]
                        sh = sh + rows[r0 + l, pl.ds(16, 16)]
                    lo.append(sl)
                    hi.append(sh)
                out_r = c * C + j
                sums_v[out_r, pl.ds(0, 16)] = (lo[0] + lo[1]) + (lo[2] + lo[3]) + lo[4]
                sums_v[out_r, pl.ds(16, 16)] = (hi[0] + hi[1]) + (hi[2] + hi[3]) + hi[4]
                return carry

            lax.fori_loop(0, C, jbody, 0)

        def pbody(p, carry):
            for sub in range(2):
                c = p * 2 + sub
                wait(c, sub)
                reduce_chunk(c, bufs[sub])
                nxt = c + 2

                @pl.when(nxt < NCH)
                def _():
                    fire(nxt, sub)

            return carry

        lax.fori_loop(0, NCH // 2, pbody, 0)
        pltpu.sync_copy(sums_v, out_hbm.at[pl.ds(wid * RPW, RPW)])

    return k(desc_flat, table)


def _lin_body(s_ref, a_ref, wm_ref, wa_ref, b_ref, o_ref):
    o_ref[...] = (
        jnp.dot(s_ref[...], wm_ref[...], preferred_element_type=jnp.float32)
        + a_ref[...] * wa_ref[...]
        + b_ref[...]
    )


def _tc_linear(sums, amounts, wm, wa, b2):
    blk = 1024
    return pl.pallas_call(
        _lin_body,
        grid=(B // blk,),
        in_specs=[
            pl.BlockSpec((blk, D), lambda i: (i, 0)),
            pl.BlockSpec((blk, 1), lambda i: (i, 0)),
            pl.BlockSpec((D, NCAT), lambda i: (0, 0)),
            pl.BlockSpec((1, NCAT), lambda i: (0, 0)),
            pl.BlockSpec((1, NCAT), lambda i: (0, 0)),
        ],
        out_specs=pl.BlockSpec((blk, NCAT), lambda i: (i, 0)),
        out_shape=jax.ShapeDtypeStruct((B, NCAT), jnp.float32),
    )(sums, amounts, wm, wa, b2)


def kernel(descriptions, amounts, table, W, b):
    desc_flat = descriptions.reshape(-1)
    sums = _sc_pool(desc_flat, table)
    wm = W[:D] * (1.0 / L)
    wa = W[D : D + 1]
    b2 = b.reshape(1, NCAT)
    return _tc_linear(sums, amounts, wm, wa, b2)
